# baseline (speedup 1.0000x reference)
import functools

import jax
import jax.numpy as jnp
from jax import lax
from jax.experimental import pallas as pl
from jax.experimental.pallas import tpu as pltpu
from jax.experimental.pallas import tpu_sc as plsc

BATCH = 16384
RANK = 32
NC, NS, L = 2, 16, 16
NW = NC * NS
BPW = BATCH // NW


def _cp_body(idx0_hbm, idx1_hbm, idx2_hbm, lamb_hbm, f0_hbm, f1_hbm, f2_hbm,
             out_hbm, i0_v, i1_v, i2_v, g0_v, g1_v, g2_v, lamb_v, out_v, sem):
    wid = lax.axis_index("s") * NC + lax.axis_index("c")
    base = wid * BPW

    pltpu.sync_copy(idx0_hbm.at[pl.ds(base, BPW)], i0_v)
    pltpu.sync_copy(idx1_hbm.at[pl.ds(base, BPW)], i1_v)
    pltpu.sync_copy(idx2_hbm.at[pl.ds(base, BPW)], i2_v)
    pltpu.sync_copy(lamb_hbm, lamb_v)

    c0 = pltpu.make_async_copy(f0_hbm.at[i0_v], g0_v, sem)
    c1 = pltpu.make_async_copy(f1_hbm.at[i1_v], g1_v, sem)
    c2 = pltpu.make_async_copy(f2_hbm.at[i2_v], g2_v, sem)
    c0.start()
    c1.start()
    c2.start()
    c0.wait()
    c1.wait()
    c2.wait()

    lanes = lax.iota(jnp.int32, L)

    def group(g, carry):
        rows = lanes + g * L
        acc = jnp.zeros((L,), jnp.float32)
        for j in range(RANK):
            col = jnp.full((L,), j, jnp.int32)
            v0 = plsc.load_gather(g0_v, [rows, col])
            v1 = plsc.load_gather(g1_v, [rows, col])
            v2 = plsc.load_gather(g2_v, [rows, col])
            lj = plsc.load_gather(lamb_v, [col])
            acc = acc + (v0 * v1) * (v2 * lj)
        out_v[pl.ds(g * L, L)] = acc
        return carry

    lax.fori_loop(0, BPW // L, group, 0)

    pltpu.sync_copy(out_v, out_hbm.at[pl.ds(base, BPW)])


_cp_kernel = functools.partial(
    pl.kernel,
    out_type=jax.ShapeDtypeStruct((BATCH,), jnp.float32),
    mesh=plsc.VectorSubcoreMesh(core_axis_name="c", subcore_axis_name="s",
                                num_cores=NC, num_subcores=NS),
    compiler_params=pltpu.CompilerParams(needs_layout_passes=False,
                                         use_tc_tiling_on_sc=False),
    scratch_types=[
        pltpu.VMEM((BPW,), jnp.int32),
        pltpu.VMEM((BPW,), jnp.int32),
        pltpu.VMEM((BPW,), jnp.int32),
        pltpu.VMEM((BPW, RANK), jnp.float32),
        pltpu.VMEM((BPW, RANK), jnp.float32),
        pltpu.VMEM((BPW, RANK), jnp.float32),
        pltpu.VMEM((RANK,), jnp.float32),
        pltpu.VMEM((BPW,), jnp.float32),
        pltpu.SemaphoreType.DMA,
    ],
)(_cp_body)


def kernel(input, lamb, f0, f1, f2):
    idx = input.astype(jnp.int32)
    return _cp_kernel(idx[:, 0], idx[:, 1], idx[:, 2], lamb, f0, f1, f2)


# speedup vs baseline: 1.0150x; 1.0150x over previous
import functools

import jax
import jax.numpy as jnp
from jax import lax
from jax.experimental import pallas as pl
from jax.experimental.pallas import tpu as pltpu
from jax.experimental.pallas import tpu_sc as plsc

BATCH = 16384
RANK = 32
NC, NS, L = 2, 16, 16
NW = NC * NS
BPW = BATCH // NW
HALF = BPW // 2


def _cp_body(idx_hbm, lamb_hbm, f0_hbm, f1_hbm, f2_hbm, out_hbm,
             i_v, g0_v, g1_v, g2_v, lamb_v, out_v, sem0, sem1):
    wid = lax.axis_index("s") * NC + lax.axis_index("c")
    base = wid * BPW

    pltpu.sync_copy(idx_hbm.at[:, pl.ds(base, BPW)], i_v)

    copies = []
    for half, sem in ((0, sem0), (1, sem1)):
        sl = pl.ds(half * HALF, HALF)
        for k, (f_hbm, g_v) in enumerate(
                ((f0_hbm, g0_v), (f1_hbm, g1_v), (f2_hbm, g2_v))):
            c = pltpu.make_async_copy(f_hbm.at[i_v.at[k, sl]], g_v.at[sl], sem)
            c.start()
            copies.append(c)

    pltpu.sync_copy(lamb_hbm, lamb_v)
    lam = [plsc.load_gather(lamb_v, [jnp.full((L,), j, jnp.int32)])
           for j in range(RANK)]
    lanes = lax.iota(jnp.int32, L)

    def group(g, carry):
        rows = lanes + g * L
        acc = jnp.zeros((L,), jnp.float32)
        for j in range(RANK):
            col = jnp.full((L,), j, jnp.int32)
            v0 = plsc.load_gather(g0_v, [rows, col])
            v1 = plsc.load_gather(g1_v, [rows, col])
            v2 = plsc.load_gather(g2_v, [rows, col])
            acc = acc + (v0 * v1) * (v2 * lam[j])
        out_v[pl.ds(g * L, L)] = acc
        return carry

    copies[0].wait()
    copies[1].wait()
    copies[2].wait()
    lax.fori_loop(0, HALF // L, group, 0)
    copies[3].wait()
    copies[4].wait()
    copies[5].wait()
    lax.fori_loop(HALF // L, BPW // L, group, 0)

    pltpu.sync_copy(out_v, out_hbm.at[pl.ds(base, BPW)])


_cp_kernel = functools.partial(
    pl.kernel,
    out_type=jax.ShapeDtypeStruct((BATCH,), jnp.float32),
    mesh=plsc.VectorSubcoreMesh(core_axis_name="c", subcore_axis_name="s",
                                num_cores=NC, num_subcores=NS),
    compiler_params=pltpu.CompilerParams(needs_layout_passes=False,
                                         use_tc_tiling_on_sc=False),
    scratch_types=[
        pltpu.VMEM((3, BPW), jnp.int32),
        pltpu.VMEM((BPW, RANK), jnp.float32),
        pltpu.VMEM((BPW, RANK), jnp.float32),
        pltpu.VMEM((BPW, RANK), jnp.float32),
        pltpu.VMEM((RANK,), jnp.float32),
        pltpu.VMEM((BPW,), jnp.float32),
        pltpu.SemaphoreType.DMA,
        pltpu.SemaphoreType.DMA,
    ],
)(_cp_body)


def kernel(input, lamb, f0, f1, f2):
    idx_t = input.T.astype(jnp.int32)
    return _cp_kernel(idx_t, lamb, f0, f1, f2)
